# baseline (speedup 1.0000x reference)
import functools

import jax
import jax.numpy as jnp
from jax import lax
from jax.experimental import pallas as pl
from jax.experimental.pallas import tpu as pltpu
from jax.experimental.pallas import tpu_sc as plsc

L = 16
NC = 2
NS = 16
NW = NC * NS

D = 128
G = D // L
NBUF = 4
RU = 10
KSC = 2176
BB = 32


def _sc_permute(x, perm, nsc):
    B, S, _ = x.shape
    batches_per_w = nsc // NW
    nt = batches_per_w // NBUF
    mesh = plsc.VectorSubcoreMesh(core_axis_name="c", subcore_axis_name="s")

    @functools.partial(
        pl.kernel,
        mesh=mesh,
        compiler_params=pltpu.CompilerParams(needs_layout_passes=False),
        out_type=jax.ShapeDtypeStruct((nsc, S, D), jnp.float32),
        scratch_types=(
            [pltpu.VMEM((D,), jnp.int32)]
            + [pltpu.VMEM((S, D), jnp.float32) for _ in range(2 * NBUF)]
            + [pltpu.SemaphoreType.DMA for _ in range(2 * NBUF)]
        ),
    )
    def k(x_hbm, perm_hbm, out_hbm, perm_v,
          i0, i1, i2, i3, o0, o1, o2, o3,
          si0, si1, si2, si3, so0, so1, so2, so3):
        ins = (i0, i1, i2, i3)
        outs = (o0, o1, o2, o3)
        sins = (si0, si1, si2, si3)
        souts = (so0, so1, so2, so3)

        wid = lax.axis_index("s") * NC + lax.axis_index("c")
        bbase = wid * batches_per_w
        pltpu.sync_copy(perm_hbm, perm_v)
        cols = tuple(perm_v[pl.ds(g * L, L)] for g in range(G))

        def cp_in(t, b):
            return pltpu.make_async_copy(x_hbm.at[bbase + t], ins[b], sins[b])

        def cp_out(t, b):
            return pltpu.make_async_copy(outs[b], out_hbm.at[bbase + t], souts[b])

        for b in range(NBUF):
            cp_in(b, b).start()

        def permute(inb, oub):
            @plsc.parallel_loop(0, S, unroll=RU)
            def body(r):
                rv = jnp.full((L,), r, dtype=jnp.int32)
                for g in range(G):
                    oub[r, pl.ds(g * L, L)] = plsc.load_gather(inb, [rv, cols[g]])

        def outer(t4, c):
            for b in range(NBUF):
                t = t4 * NBUF + b
                cp_in(t, b).wait()

                @pl.when(t4 > 0)
                def _():
                    cp_out(t - NBUF, b).wait()

                permute(ins[b], outs[b])
                cp_out(t, b).start()

                @pl.when(t4 + 1 < nt)
                def _():
                    cp_in(t + NBUF, b).start()

            return c

        lax.fori_loop(0, nt, outer, 0)
        for b in range(NBUF):
            cp_out((nt - 1) * NBUF + b, b).wait()

    return k(x, perm)


def _tc_permute_tail(x, perm, ksc):
    B, S, _ = x.shape
    k0 = ksc // BB

    def body(x_ref, perm_ref, o_ref):
        iota = lax.broadcasted_iota(jnp.int32, (D, D), 0)
        onehot = (iota == perm_ref[...][None, :]).astype(jnp.float32)
        o_ref[...] = lax.dot_general(
            x_ref[...], onehot, (((2,), (0,)), ((), ())),
            preferred_element_type=jnp.float32)

    return pl.pallas_call(
        body,
        grid=((B - ksc) // BB,),
        in_specs=[
            pl.BlockSpec((BB, S, D), lambda i: (i + k0, 0, 0)),
            pl.BlockSpec((D,), lambda i: (0,)),
        ],
        out_specs=pl.BlockSpec((BB, S, D), lambda i: (i + k0, 0, 0)),
        out_shape=jax.ShapeDtypeStruct((B, S, D), jnp.float32),
    )(x, perm)


@jax.jit
def _permute(x, perm):
    sc_out = _sc_permute(x, perm, KSC)
    tc_full = _tc_permute_tail(x, perm, KSC)
    return lax.dynamic_update_slice(tc_full, sc_out, (0, 0, 0))


def kernel(input, permutation):
    return _permute(input, permutation.astype(jnp.int32))
